# baseline (speedup 1.0000x reference)
import functools

import jax
import jax.numpy as jnp
from jax.experimental import pallas as pl
from jax.experimental.pallas import tpu as pltpu

_EPS = 1e-8


def _cos_loss_kernel(a_ref, b_ref, o_ref, *, scale, tiles):
    parts = None
    for g in range(tiles):
        a = a_ref[g]
        b = b_ref[g]
        num = jnp.sum(a * b, axis=-1)
        na2 = jnp.sum(a * a, axis=-1)
        nb2 = jnp.sum(b * b, axis=-1)
        denom2 = jnp.maximum(na2, _EPS * _EPS) * jnp.maximum(nb2, _EPS * _EPS)
        part = num * jax.lax.rsqrt(denom2)
        parts = part if parts is None else parts + part
    s = jnp.sum(parts)

    i = pl.program_id(0)
    nsteps = pl.num_programs(0)

    @pl.when(i == 0)
    def _():
        o_ref[...] = jnp.zeros((1, 1), jnp.float32)

    o_ref[...] = o_ref[...] + s

    @pl.when(i == nsteps - 1)
    def _():
        o_ref[...] = o_ref[...] * scale


def kernel(logits, logits1, p0first, p0sec, orixyz):
    b, n, c = logits.shape
    rows = b * n
    tile_rows = c
    groups = rows // tile_rows
    a3 = logits.reshape(groups, tile_rows, c)
    b3 = logits1.reshape(groups, tile_rows, c)

    grid = 8
    chunk = groups // grid

    out = pl.pallas_call(
        functools.partial(_cos_loss_kernel, scale=-1.0 / rows, tiles=chunk),
        grid=(grid,),
        in_specs=[
            pl.BlockSpec((chunk, tile_rows, c), lambda i: (i, 0, 0)),
            pl.BlockSpec((chunk, tile_rows, c), lambda i: (i, 0, 0)),
        ],
        out_specs=pl.BlockSpec((1, 1), lambda i: (0, 0)),
        out_shape=jax.ShapeDtypeStruct((1, 1), jnp.float32),
    )(a3, b3)
    return out[0, 0]


# speedup vs baseline: 1.0414x; 1.0414x over previous
import functools

import jax
import jax.numpy as jnp
from jax.experimental import pallas as pl
from jax.experimental.pallas import tpu as pltpu

_EPS = 1e-8


def _cos_loss_kernel(a_ref, b_ref, o_ref, *, scale, tiles):
    parts = None
    for g in range(tiles):
        a = a_ref[g]
        b = b_ref[g]
        num = jnp.sum(a * b, axis=-1)
        na2 = jnp.sum(a * a, axis=-1)
        nb2 = jnp.sum(b * b, axis=-1)
        denom2 = jnp.maximum(na2, _EPS * _EPS) * jnp.maximum(nb2, _EPS * _EPS)
        part = num * jax.lax.rsqrt(denom2)
        parts = part if parts is None else parts + part
    s = jnp.sum(parts)

    i = pl.program_id(0)
    nsteps = pl.num_programs(0)

    @pl.when(i == 0)
    def _():
        o_ref[...] = jnp.zeros((1, 1), jnp.float32)

    o_ref[...] = o_ref[...] + s

    @pl.when(i == nsteps - 1)
    def _():
        o_ref[...] = o_ref[...] * scale


def kernel(logits, logits1, p0first, p0sec, orixyz):
    b, n, c = logits.shape
    rows = b * n
    tile_rows = c
    groups = rows // tile_rows
    a3 = logits.reshape(groups, tile_rows, c)
    b3 = logits1.reshape(groups, tile_rows, c)

    grid = 4
    chunk = groups // grid

    out = pl.pallas_call(
        functools.partial(_cos_loss_kernel, scale=-1.0 / rows, tiles=chunk),
        grid=(grid,),
        in_specs=[
            pl.BlockSpec((chunk, tile_rows, c), lambda i: (i, 0, 0)),
            pl.BlockSpec((chunk, tile_rows, c), lambda i: (i, 0, 0)),
        ],
        out_specs=pl.BlockSpec((1, 1), lambda i: (0, 0)),
        out_shape=jax.ShapeDtypeStruct((1, 1), jnp.float32),
    )(a3, b3)
    return out[0, 0]
